# baseline (speedup 1.0000x reference)
import functools

import jax
import jax.numpy as jnp
from jax import lax
from jax.experimental import pallas as pl
from jax.experimental.pallas import tpu as pltpu
from jax.experimental.pallas import tpu_sc as plsc

NW = 32
CHUNK = 128
LANES = 16
NBUF = 5
PITCH = CHUNK + 1
RPI = 4
PACK = 4


def _build(vocab, dim, hist):
    mesh = plsc.VectorSubcoreMesh(core_axis_name="c", subcore_axis_name="s")
    line_w = PACK * dim

    @functools.partial(
        pl.kernel,
        mesh=mesh,
        compiler_params=pltpu.CompilerParams(
            use_tc_tiling_on_sc=False, needs_layout_passes=False
        ),
        out_type=jax.ShapeDtypeStruct((hist, dim // 8, NW, 8, CHUNK), jnp.float32),
        scratch_types=[
            pltpu.VMEM((hist, CHUNK), jnp.int32),
            pltpu.VMEM((hist, CHUNK), jnp.int32),
            pltpu.VMEM((NBUF, CHUNK, line_w), jnp.float32),
            pltpu.VMEM((NBUF, dim // 8, 8, PITCH), jnp.float32),
        ]
        + [pltpu.SemaphoreType.DMA] * (2 * NBUF),
    )
    def fractal_gather(
        idx_hbm, table_hbm, out_hbm, idx_v, line_v, in_v, out_v, *sems
    ):
        gsems, ssems = sems[:NBUF], sems[NBUF:]
        wid = lax.axis_index("s") * 2 + lax.axis_index("c")
        pltpu.sync_copy(idx_hbm.at[wid], idx_v)
        iot = lax.iota(jnp.int32, LANES)
        dbs, sss = [], []
        for s in range(0, dim, LANES):
            dvec = lax.iota(jnp.int32, LANES) + s
            dbs.append(dvec // 8)
            sss.append(dvec % 8)

        def split_lines(h, c2):
            for l0 in range(0, CHUNK, LANES):
                tok = idx_v[h, pl.ds(l0, LANES)]
                line_v[h, pl.ds(l0, LANES)] = tok // PACK
                idx_v[h, pl.ds(l0, LANES)] = (tok % PACK) * dim
            return c2

        lax.fori_loop(0, hist, split_lines, 0)

        def start_chunk(h, b):
            pltpu.async_copy(table_hbm.at[line_v.at[h]], in_v.at[b], gsems[b])

        for b in range(NBUF):
            start_chunk(b, b)

        def outer(g, carry):
            for b in range(NBUF):
                h = g * NBUF + b
                pltpu.make_async_copy(
                    table_hbm.at[line_v.at[h]], in_v.at[b], gsems[b]
                ).wait()
                @pl.when(g > 0)
                def _wait_store():
                    pltpu.make_async_copy(
                        out_v.at[b, :, :, pl.ds(0, CHUNK)],
                        out_hbm.at[0, :, 0],
                        ssems[b],
                    ).wait()

                @plsc.parallel_loop(0, CHUNK // RPI, step=1)
                def _compute(t4):
                    sgroup = idx_v[h, pl.ds((t4 // RPI) * LANES, LANES)]
                    cs, tvecs = [], []
                    for rr in range(RPI):
                        t = t4 * RPI + rr
                        tvecs.append(jnp.full((LANES,), t, jnp.int32))
                        lane = (t4 % RPI) * RPI + rr
                        basevec = sgroup.at[jnp.full((LANES,), lane, jnp.int32)].get(
                            mode="promise_in_bounds"
                        )
                        for s in range(0, dim, LANES):
                            cs.append(
                                plsc.load_gather(
                                    in_v.at[b],
                                    [tvecs[rr], basevec + (iot + s)],
                                )
                            )
                    zs = list(cs)
                    for _ in range(9):
                        zs = [z * z + c for z, c in zip(zs, cs)]
                    i = 0
                    for rr in range(RPI):
                        for k in range(dim // LANES):
                            plsc.store_scatter(
                                out_v.at[b], [dbs[k], sss[k], tvecs[rr]], zs[i]
                            )
                            i += 1

                @pl.when(h + NBUF < hist)
                def _next_gather():
                    start_chunk(h + NBUF, b)

                pltpu.async_copy(
                    out_v.at[b, :, :, pl.ds(0, CHUNK)],
                    out_hbm.at[h, :, wid],
                    ssems[b],
                )
            return carry

        lax.fori_loop(0, hist // NBUF, outer, 0)
        for b in range(NBUF):
            pltpu.make_async_copy(
                out_v.at[b, :, :, pl.ds(0, CHUNK)], out_hbm.at[0, :, 0], ssems[b]
            ).wait()

    return fractal_gather


def kernel(token_id, weights):
    batch, hist = token_id.shape
    vocab, dim = weights.shape
    assert batch == NW * CHUNK and dim % LANES == 0 and hist % NBUF == 0
    assert vocab % PACK == 0
    idx = token_id.reshape(NW, CHUNK, hist).transpose(0, 2, 1).astype(jnp.int32)
    table = weights.reshape(vocab // PACK, PACK * dim)
    out5 = _build(vocab, dim, hist)(idx, table)
    out = out5.transpose(2, 4, 0, 1, 3).reshape(batch, hist, dim)
    return out


# speedup vs baseline: 1.0018x; 1.0018x over previous
import functools

import jax
import jax.numpy as jnp
from jax import lax
from jax.experimental import pallas as pl
from jax.experimental.pallas import tpu as pltpu
from jax.experimental.pallas import tpu_sc as plsc

NW = 32
CHUNK = 128
LANES = 16
NBUF = 5
PITCH = CHUNK + 1
RPI = 4
PACK = 4


def _build(vocab, dim, hist):
    mesh = plsc.VectorSubcoreMesh(core_axis_name="c", subcore_axis_name="s")
    line_w = PACK * dim

    @functools.partial(
        pl.kernel,
        mesh=mesh,
        compiler_params=pltpu.CompilerParams(
            use_tc_tiling_on_sc=False, needs_layout_passes=False
        ),
        out_type=jax.ShapeDtypeStruct((hist, dim // 8, NW, 8, CHUNK), jnp.float32),
        scratch_types=[
            pltpu.VMEM((hist, CHUNK), jnp.int32),
            pltpu.VMEM((hist, CHUNK), jnp.int32),
            pltpu.VMEM((NBUF, CHUNK, line_w), jnp.float32),
            pltpu.VMEM((NBUF, dim // 8, 8, PITCH), jnp.float32),
        ]
        + [pltpu.SemaphoreType.DMA] * (2 * NBUF),
    )
    def fractal_gather(
        idx_hbm, table_hbm, out_hbm, idx_v, line_v, in_v, out_v, *sems
    ):
        gsems, ssems = sems[:NBUF], sems[NBUF:]
        wid = lax.axis_index("s") * 2 + lax.axis_index("c")
        pltpu.sync_copy(idx_hbm.at[wid], idx_v)
        iot = lax.iota(jnp.int32, LANES)
        dbs, sss = [], []
        for s in range(0, dim, LANES):
            dvec = lax.iota(jnp.int32, LANES) + s
            dbs.append(dvec // 8)
            sss.append(dvec % 8)

        def split_lines(h, c2):
            for l0 in range(0, CHUNK, LANES):
                tok = idx_v[h, pl.ds(l0, LANES)]
                line_v[h, pl.ds(l0, LANES)] = tok // PACK
                idx_v[h, pl.ds(l0, LANES)] = (tok % PACK) * dim
            return c2

        lax.fori_loop(0, hist, split_lines, 0)

        def start_chunk(h, b):
            pltpu.async_copy(table_hbm.at[line_v.at[h]], in_v.at[b], gsems[b])

        for b in range(NBUF):
            start_chunk(b, b)

        def outer(g, carry):
            for b in range(NBUF):
                h = g * NBUF + b
                pltpu.make_async_copy(
                    table_hbm.at[line_v.at[h]], in_v.at[b], gsems[b]
                ).wait()
                @pl.when(g > 0)
                def _wait_store():
                    pltpu.make_async_copy(
                        out_v.at[b, :, :, pl.ds(0, CHUNK)],
                        out_hbm.at[0, :, 0],
                        ssems[b],
                    ).wait()

                @plsc.parallel_loop(0, CHUNK // RPI, step=1)
                def _compute(t4):
                    sgroup = idx_v[h, pl.ds((t4 // RPI) * LANES, LANES)]
                    cs, tvecs = [], []
                    for rr in range(RPI):
                        t = t4 * RPI + rr
                        tvecs.append(jnp.full((LANES,), t, jnp.int32))
                        lane = (t4 % RPI) * RPI + rr
                        basevec = sgroup.at[jnp.full((LANES,), lane, jnp.int32)].get(
                            mode="promise_in_bounds"
                        )
                        for s in range(0, dim, LANES):
                            cs.append(
                                plsc.load_gather(
                                    in_v.at[b],
                                    [tvecs[rr], basevec + (iot + s)],
                                )
                            )
                    zs = list(cs)
                    for _ in range(9):
                        zs = [z * z + c for z, c in zip(zs, cs)]
                    i = 0
                    for rr in range(RPI):
                        for k in range(dim // LANES):
                            plsc.store_scatter(
                                out_v.at[b], [dbs[k], sss[k], tvecs[rr]], zs[i]
                            )
                            i += 1

                @pl.when(h + NBUF < hist)
                def _next_gather():
                    start_chunk(h + NBUF, b)

                pltpu.async_copy(
                    out_v.at[b, :, :, pl.ds(0, CHUNK)],
                    out_hbm.at[h, :, wid],
                    ssems[b],
                )
            return carry

        lax.fori_loop(0, hist // NBUF, outer, 0)
        for b in range(NBUF):
            pltpu.make_async_copy(
                out_v.at[b, :, :, pl.ds(0, CHUNK)], out_hbm.at[0, :, 0], ssems[b]
            ).wait()

    return fractal_gather


def kernel(token_id, weights):
    batch, hist = token_id.shape
    vocab, dim = weights.shape
    assert batch == NW * CHUNK and dim % LANES == 0 and hist % NBUF == 0
    assert vocab % PACK == 0
    idx = token_id.reshape(NW, CHUNK, hist).transpose(0, 2, 1).astype(jnp.int32)
    table = lax.optimization_barrier(weights.reshape(vocab // PACK, PACK * dim))
    out5 = _build(vocab, dim, hist)(idx, table)
    out = out5.transpose(2, 4, 0, 1, 3).reshape(batch, hist, dim)
    return out


# speedup vs baseline: 1.0435x; 1.0416x over previous
import functools

import jax
import jax.numpy as jnp
from jax import lax
from jax.experimental import pallas as pl
from jax.experimental.pallas import tpu as pltpu
from jax.experimental.pallas import tpu_sc as plsc

NW = 32
CHUNK = 128
LANES = 16
NBUF = 5
PITCH = CHUNK + 1
RPI = 4


def _build(vocab, dim, hist):
    mesh = plsc.VectorSubcoreMesh(core_axis_name="c", subcore_axis_name="s")

    @functools.partial(
        pl.kernel,
        mesh=mesh,
        compiler_params=pltpu.CompilerParams(
            use_tc_tiling_on_sc=False, needs_layout_passes=False
        ),
        out_type=jax.ShapeDtypeStruct((hist, dim // 8, NW, 8, CHUNK), jnp.float32),
        scratch_types=[
            pltpu.VMEM((hist, CHUNK), jnp.int32),
            pltpu.VMEM((NBUF, CHUNK, dim), jnp.float32),
            pltpu.VMEM((NBUF, dim // 8, 8, PITCH), jnp.float32),
        ]
        + [pltpu.SemaphoreType.DMA] * (2 * NBUF),
    )
    def fractal_gather(idx_hbm, table_hbm, out_hbm, idx_v, in_v, out_v, *sems):
        gsems, ssems = sems[:NBUF], sems[NBUF:]
        wid = lax.axis_index("s") * 2 + lax.axis_index("c")
        pltpu.sync_copy(idx_hbm.at[wid], idx_v)
        dbs, sss = [], []
        for s in range(0, dim, LANES):
            dvec = lax.iota(jnp.int32, LANES) + s
            dbs.append(dvec // 8)
            sss.append(dvec % 8)
        for b in range(NBUF):
            pltpu.async_copy(table_hbm.at[idx_v.at[b]], in_v.at[b], gsems[b])

        def outer(g, carry):
            for b in range(NBUF):
                h = g * NBUF + b
                pltpu.make_async_copy(
                    table_hbm.at[idx_v.at[h]], in_v.at[b], gsems[b]
                ).wait()

                @pl.when(g > 0)
                def _wait_store():
                    pltpu.make_async_copy(
                        out_v.at[b, :, :, pl.ds(0, CHUNK)],
                        out_hbm.at[0, :, 0],
                        ssems[b],
                    ).wait()

                @plsc.parallel_loop(0, CHUNK // RPI, step=1)
                def _compute(t4):
                    cs, tvecs = [], []
                    for rr in range(RPI):
                        t = t4 * RPI + rr
                        tvecs.append(jnp.full((LANES,), t, jnp.int32))
                        for s in range(0, dim, LANES):
                            cs.append(in_v[b, t, pl.ds(s, LANES)])
                    zs = list(cs)
                    for _ in range(9):
                        zs = [z * z + c for z, c in zip(zs, cs)]
                    i = 0
                    for rr in range(RPI):
                        for k in range(dim // LANES):
                            plsc.store_scatter(
                                out_v.at[b], [dbs[k], sss[k], tvecs[rr]], zs[i]
                            )
                            i += 1

                @pl.when(h + NBUF < hist)
                def _next_gather():
                    pltpu.async_copy(
                        table_hbm.at[idx_v.at[h + NBUF]], in_v.at[b], gsems[b]
                    )

                pltpu.async_copy(
                    out_v.at[b, :, :, pl.ds(0, CHUNK)],
                    out_hbm.at[h, :, wid],
                    ssems[b],
                )
            return carry

        lax.fori_loop(0, hist // NBUF, outer, 0)
        for b in range(NBUF):
            pltpu.make_async_copy(
                out_v.at[b, :, :, pl.ds(0, CHUNK)], out_hbm.at[0, :, 0], ssems[b]
            ).wait()

    return fractal_gather


def kernel(token_id, weights):
    batch, hist = token_id.shape
    vocab, dim = weights.shape
    assert batch == NW * CHUNK and dim % LANES == 0 and hist % NBUF == 0
    idx = token_id.reshape(NW, CHUNK, hist).transpose(0, 2, 1).astype(jnp.int32)
    out5 = _build(vocab, dim, hist)(idx, weights)
    out = out5.transpose(2, 4, 0, 1, 3).reshape(batch, hist, dim)
    return out
